# Initial kernel scaffold; baseline (speedup 1.0000x reference)
import jax
import jax.numpy as jnp
from jax.experimental import pallas as pl


def kernel(x, edge_index, edge_weight, W, b):
    raise NotImplementedError("write your pallas kernel here")


# speedup vs baseline: 1.2611x; 1.2611x over previous
import functools

import jax
import jax.numpy as jnp
from jax import lax
from jax.experimental import pallas as pl
from jax.experimental.pallas import tpu as pltpu
from jax.experimental.pallas import tpu_sc as plsc

N_NODES = 10000
N_EDGES = 160000
D_IN = 256
D_OUT = 256
DQ = 64
NC, NS, L = 2, 16, 16
E_PER_TILE = N_EDGES // NS
K = 80
NCHUNK = E_PER_TILE // K
N_PAD = 10240
ROWS_PER_TILE = N_PAD // NS


def _mm_body(x_ref, wt_ref, b_ref, o_ref):
    o_ref[0] = jnp.dot(x_ref[...], wt_ref[0],
                       preferred_element_type=jnp.float32) + b_ref[0]


def _linear_quarters(x, Wt4, b4):
    RB = 1000
    return pl.pallas_call(
        _mm_body,
        grid=(N_NODES // RB, 4),
        in_specs=[
            pl.BlockSpec((RB, D_IN), lambda i, j: (i, 0)),
            pl.BlockSpec((1, D_IN, DQ), lambda i, j: (j, 0, 0)),
            pl.BlockSpec((1, 1, DQ), lambda i, j: (j, 0, 0)),
        ],
        out_specs=pl.BlockSpec((1, RB, DQ), lambda i, j: (j, i, 0)),
        out_shape=jax.ShapeDtypeStruct((4, N_NODES, DQ), jnp.float32),
    )(x, Wt4, b4)


def _sc_aggregate(h0, h1, h2, h3, src, dst, w):
    mesh = plsc.VectorSubcoreMesh(core_axis_name="c", subcore_axis_name="s")

    @functools.partial(
        pl.kernel,
        out_type=[jax.ShapeDtypeStruct((N_PAD, DQ), jnp.float32)] * 4,
        mesh=mesh,
        scratch_types=[
            pltpu.VMEM((K,), jnp.int32),
            pltpu.VMEM((K,), jnp.int32),
            pltpu.VMEM((K,), jnp.float32),
            pltpu.VMEM((K, DQ), jnp.float32),
            pltpu.VMEM((ROWS_PER_TILE, DQ), jnp.float32),
            pltpu.VMEM_SHARED((N_PAD, DQ), jnp.float32),
            pltpu.SemaphoreType.DMA,
        ],
        compiler_params=pltpu.CompilerParams(use_tc_tiling_on_sc=False),
    )
    def k(h0_hbm, h1_hbm, h2_hbm, h3_hbm, src_hbm, dst_hbm, w_hbm,
          o0_hbm, o1_hbm, o2_hbm, o3_hbm,
          idx_v, dst_v, w_v, rows_v, buf_v, acc_sh, sem):
        c = lax.axis_index("c")
        s = lax.axis_index("s")
        r0 = s * ROWS_PER_TILE

        def zrow(i, carry):
            for j in range(DQ // L):
                buf_v[i, pl.ds(j * L, L)] = jnp.zeros((L,), jnp.float32)
            return carry
        lax.fori_loop(0, ROWS_PER_TILE, zrow, 0)

        def pipeline(h_hbm, o_hbm):
            pltpu.sync_copy(buf_v, acc_sh.at[pl.ds(r0, ROWS_PER_TILE)])
            plsc.subcore_barrier()
            base0 = s * E_PER_TILE

            def chunk(t, carry):
                base = base0 + t * K
                pltpu.sync_copy(src_hbm.at[pl.ds(base, K)], idx_v)
                pltpu.sync_copy(dst_hbm.at[pl.ds(base, K)], dst_v)
                pltpu.sync_copy(w_hbm.at[pl.ds(base, K)], w_v)
                pltpu.async_copy(h_hbm.at[idx_v], rows_v, sem).wait()

                def mul(g, cc):
                    wv16 = w_v[pl.ds(g * L, L)]
                    for e in range(L):
                        wv = wv16[e]
                        i = g * L + e
                        for j in range(DQ // L):
                            sl = pl.ds(j * L, L)
                            rows_v[i, sl] = rows_v[i, sl] * wv
                    return cc
                lax.fori_loop(0, K // L, mul, 0)
                pltpu.sync_copy(rows_v, acc_sh.at[dst_v], add=True)
                return carry
            lax.fori_loop(0, NCHUNK, chunk, 0)
            plsc.subcore_barrier()
            pltpu.sync_copy(acc_sh.at[pl.ds(r0, ROWS_PER_TILE)], buf_v)
            pltpu.sync_copy(buf_v, o_hbm.at[pl.ds(r0, ROWS_PER_TILE)])
            lax.fori_loop(0, ROWS_PER_TILE, zrow, 0)

        @pl.when(c == 0)
        def _():
            pipeline(h0_hbm, o0_hbm)
            pipeline(h1_hbm, o1_hbm)

        @pl.when(c == 1)
        def _():
            pipeline(h2_hbm, o2_hbm)
            pipeline(h3_hbm, o3_hbm)

    return k(h0, h1, h2, h3, src, dst, w)


def kernel(x, edge_index, edge_weight, W, b):
    src = edge_index[0].astype(jnp.int32)
    dst = edge_index[1].astype(jnp.int32)
    wt4 = jnp.transpose(W.T.reshape(D_IN, 4, DQ), (1, 0, 2))
    h4 = _linear_quarters(x, wt4, b.reshape(4, 1, DQ))
    outs = _sc_aggregate(h4[0], h4[1], h4[2], h4[3], src, dst, edge_weight)
    return jnp.concatenate([o[:N_NODES] for o in outs], axis=1)


# speedup vs baseline: 2.1500x; 1.7049x over previous
import functools

import jax
import jax.numpy as jnp
from jax import lax
from jax.experimental import pallas as pl
from jax.experimental.pallas import tpu as pltpu
from jax.experimental.pallas import tpu_sc as plsc

N_NODES = 10000
N_EDGES = 160000
D_IN = 256
D_OUT = 256
DQ = 64
NC, NS, L = 2, 16, 16
K = 128
E_PAD = 163840
E_PER_TILE = E_PAD // NS
NCHUNK = E_PER_TILE // K
NPAIR = NCHUNK // 2
N_PAD = 10240
ROWS_PER_TILE = N_PAD // NS


def _mm_body(x_ref, wt_ref, b_ref, o_ref):
    o_ref[0] = jnp.dot(x_ref[...], wt_ref[0],
                       preferred_element_type=jnp.float32) + b_ref[0]


def _linear_quarters(x, Wt4, b4):
    RB = 1000
    return pl.pallas_call(
        _mm_body,
        grid=(N_NODES // RB, 4),
        in_specs=[
            pl.BlockSpec((RB, D_IN), lambda i, j: (i, 0)),
            pl.BlockSpec((1, D_IN, DQ), lambda i, j: (j, 0, 0)),
            pl.BlockSpec((1, 1, DQ), lambda i, j: (j, 0, 0)),
        ],
        out_specs=pl.BlockSpec((1, RB, DQ), lambda i, j: (j, i, 0)),
        out_shape=jax.ShapeDtypeStruct((4, N_NODES, DQ), jnp.float32),
    )(x, Wt4, b4)


def _sc_aggregate(h0, h1, h2, h3, src3, dst3, w2):
    mesh = plsc.VectorSubcoreMesh(core_axis_name="c", subcore_axis_name="s")

    @functools.partial(
        pl.kernel,
        out_type=[jax.ShapeDtypeStruct((N_PAD, DQ), jnp.float32)] * 4,
        mesh=mesh,
        scratch_types=[
            pltpu.VMEM((NCHUNK, K), jnp.int32),
            pltpu.VMEM((NCHUNK, K), jnp.int32),
            pltpu.VMEM((E_PER_TILE,), jnp.float32),
            pltpu.VMEM((K, DQ), jnp.float32),
            pltpu.VMEM((K, DQ), jnp.float32),
            pltpu.VMEM((ROWS_PER_TILE, DQ), jnp.float32),
            pltpu.VMEM_SHARED((N_PAD, DQ), jnp.float32),
            pltpu.SemaphoreType.DMA,
            pltpu.SemaphoreType.DMA,
        ],
        compiler_params=pltpu.CompilerParams(use_tc_tiling_on_sc=False),
    )
    def k(h0_hbm, h1_hbm, h2_hbm, h3_hbm, src_hbm, dst_hbm, w_hbm,
          o0_hbm, o1_hbm, o2_hbm, o3_hbm,
          src_v, dst_v, w_v, rows0, rows1, buf_v, acc_sh, sem0, sem1):
        c = lax.axis_index("c")
        s = lax.axis_index("s")
        r0 = s * ROWS_PER_TILE

        pltpu.sync_copy(src_hbm.at[s], src_v)
        pltpu.sync_copy(dst_hbm.at[s], dst_v)
        pltpu.sync_copy(w_hbm.at[s], w_v)

        def zrow(i, carry):
            for j in range(DQ // L):
                buf_v[i, pl.ds(j * L, L)] = jnp.zeros((L,), jnp.float32)
            return carry
        lax.fori_loop(0, ROWS_PER_TILE, zrow, 0)

        def pipeline(h_hbm, o_hbm):
            pltpu.sync_copy(buf_v, acc_sh.at[pl.ds(r0, ROWS_PER_TILE)])
            plsc.subcore_barrier()

            def process(t, rows):
                def mul(g, cc):
                    wv16 = w_v[pl.ds(t * K + g * L, L)]
                    for e in range(L):
                        wv = wv16[e]
                        i = g * L + e
                        for j in range(DQ // L):
                            sl = pl.ds(j * L, L)
                            rows[i, sl] = rows[i, sl] * wv
                    return cc
                lax.fori_loop(0, K // L, mul, 0)
                pltpu.sync_copy(rows, acc_sh.at[dst_v.at[t]], add=True)

            pltpu.async_copy(h_hbm.at[src_v.at[0]], rows0, sem0)

            def pair(u, carry):
                t0 = 2 * u
                d1 = pltpu.async_copy(h_hbm.at[src_v.at[t0 + 1]], rows1, sem1)
                pltpu.make_async_copy(
                    h_hbm.at[src_v.at[t0]], rows0, sem0).wait()
                process(t0, rows0)

                @pl.when(u + 1 < NPAIR)
                def _():
                    pltpu.async_copy(h_hbm.at[src_v.at[t0 + 2]], rows0, sem0)
                d1.wait()
                process(t0 + 1, rows1)
                return carry
            lax.fori_loop(0, NPAIR, pair, 0)
            plsc.subcore_barrier()
            pltpu.sync_copy(acc_sh.at[pl.ds(r0, ROWS_PER_TILE)], buf_v)
            pltpu.sync_copy(buf_v, o_hbm.at[pl.ds(r0, ROWS_PER_TILE)])
            lax.fori_loop(0, ROWS_PER_TILE, zrow, 0)

        @pl.when(c == 0)
        def _():
            pipeline(h0_hbm, o0_hbm)
            pipeline(h1_hbm, o1_hbm)

        @pl.when(c == 1)
        def _():
            pipeline(h2_hbm, o2_hbm)
            pipeline(h3_hbm, o3_hbm)

    return k(h0, h1, h2, h3, src3, dst3, w2)


def kernel(x, edge_index, edge_weight, W, b):
    src = edge_index[0].astype(jnp.int32)
    dst = edge_index[1].astype(jnp.int32)
    pad = E_PAD - N_EDGES
    zi = jnp.zeros((pad,), jnp.int32)
    src3 = jnp.concatenate([src, zi]).reshape(NS, NCHUNK, K)
    dst3 = jnp.concatenate([dst, zi]).reshape(NS, NCHUNK, K)
    w2 = jnp.concatenate(
        [edge_weight, jnp.zeros((pad,), jnp.float32)]).reshape(NS, E_PER_TILE)
    wt4 = jnp.transpose(W.T.reshape(D_IN, 4, DQ), (1, 0, 2))
    h4 = _linear_quarters(x, wt4, b.reshape(4, 1, DQ))
    outs = _sc_aggregate(h4[0], h4[1], h4[2], h4[3], src3, dst3, w2)
    return jnp.concatenate([o[:N_NODES] for o in outs], axis=1)


# speedup vs baseline: 2.9749x; 1.3837x over previous
import functools

import jax
import jax.numpy as jnp
from jax import lax
from jax.experimental import pallas as pl
from jax.experimental.pallas import tpu as pltpu
from jax.experimental.pallas import tpu_sc as plsc

N_NODES = 10000
N_EDGES = 160000
D_IN = 256
D_OUT = 256
DQ = 64
NC, NS, L = 2, 16, 16
K = 128
E_PAD = 163840
E_PER_TILE = E_PAD // NS
NCHUNK = E_PER_TILE // K
NPAIR = NCHUNK // 2
N_PAD = 10240
ROWS_PER_TILE = N_PAD // NS
RB_CP = 128
N_CP = ROWS_PER_TILE // RB_CP


def _mm_body(x_ref, wt_ref, b_ref, o_ref):
    o_ref[0] = jnp.dot(x_ref[...], wt_ref[0],
                       preferred_element_type=jnp.float32) + b_ref[0]


def _linear_quarters(x, Wt4, b4):
    RB = 1000
    return pl.pallas_call(
        _mm_body,
        grid=(N_NODES // RB, 4),
        in_specs=[
            pl.BlockSpec((RB, D_IN), lambda i, j: (i, 0)),
            pl.BlockSpec((1, D_IN, DQ), lambda i, j: (j, 0, 0)),
            pl.BlockSpec((1, 1, DQ), lambda i, j: (j, 0, 0)),
        ],
        out_specs=pl.BlockSpec((1, RB, DQ), lambda i, j: (j, i, 0)),
        out_shape=jax.ShapeDtypeStruct((4, N_NODES, DQ), jnp.float32),
    )(x, Wt4, b4)


def _sc_aggregate(h4, src3, dst3, w2):
    mesh = plsc.VectorSubcoreMesh(core_axis_name="c", subcore_axis_name="s")

    @functools.partial(
        pl.kernel,
        out_type=[jax.ShapeDtypeStruct((N_PAD, DQ), jnp.float32)] * 4,
        mesh=mesh,
        scratch_types=[
            pltpu.VMEM((NCHUNK, K), jnp.int32),
            pltpu.VMEM((NCHUNK, K), jnp.int32),
            pltpu.VMEM((E_PER_TILE,), jnp.float32),
            pltpu.VMEM((K, DQ), jnp.float32),
            pltpu.VMEM((K, DQ), jnp.float32),
            pltpu.VMEM((K, DQ), jnp.float32),
            pltpu.VMEM((K, DQ), jnp.float32),
            pltpu.VMEM((RB_CP, DQ), jnp.float32),
            pltpu.VMEM_SHARED((N_PAD, DQ), jnp.float32),
            pltpu.SemaphoreType.DMA,
            pltpu.SemaphoreType.DMA,
            pltpu.SemaphoreType.DMA,
            pltpu.SemaphoreType.DMA,
        ],
        compiler_params=pltpu.CompilerParams(use_tc_tiling_on_sc=False),
    )
    def k(h0_hbm, h1_hbm, h2_hbm, h3_hbm, src_hbm, dst_hbm, w_hbm,
          o0_hbm, o1_hbm, o2_hbm, o3_hbm,
          src_v, dst_v, w_v, gbuf0, gbuf1, sbuf0, sbuf1, buf_v, acc_sh,
          gsem0, gsem1, ssem0, ssem1):
        c = lax.axis_index("c")
        s = lax.axis_index("s")
        r0 = s * ROWS_PER_TILE

        pltpu.sync_copy(src_hbm.at[s], src_v)
        pltpu.sync_copy(dst_hbm.at[s], dst_v)
        pltpu.sync_copy(w_hbm.at[s], w_v)

        def zrow(i, carry):
            for j in range(DQ // L):
                buf_v[i, pl.ds(j * L, L)] = jnp.zeros((L,), jnp.float32)
            return carry
        lax.fori_loop(0, RB_CP, zrow, 0)

        def process(t, gbuf, sbuf):
            def mul(g, cc):
                wv16 = w_v[pl.ds(t * K + g * L, L)]
                for e in range(L):
                    wb = wv16[e]
                    i = g * L + e
                    for j in range(DQ // L):
                        sl = pl.ds(j * L, L)
                        sbuf[i, sl] = gbuf[i, sl] * wb
                return cc
            lax.fori_loop(0, K // L, mul, 0)

        def pipeline(h_q, o_q):
            def zhop(i, carry):
                pltpu.sync_copy(
                    buf_v, acc_sh.at[pl.ds(r0 + i * RB_CP, RB_CP)])
                return carry
            lax.fori_loop(0, N_CP, zhop, 0)
            plsc.subcore_barrier()

            pltpu.async_copy(h_q.at[src_v.at[0]], gbuf0, gsem0)
            pltpu.async_copy(h_q.at[src_v.at[1]], gbuf1, gsem1)

            def stage(t, gbuf, sbuf, gsem, ssem, u):
                pltpu.make_async_copy(
                    h_q.at[src_v.at[t]], gbuf, gsem).wait()
                process(t, gbuf, sbuf)

                @pl.when(t + 2 < NCHUNK)
                def _():
                    pltpu.async_copy(h_q.at[src_v.at[t + 2]], gbuf, gsem)
                pltpu.sync_copy(sbuf, acc_sh.at[dst_v.at[t]], add=True)

            def pair(u, carry):
                t0 = 2 * u
                stage(t0, gbuf0, sbuf0, gsem0, ssem0, u)
                stage(t0 + 1, gbuf1, sbuf1, gsem1, ssem1, u)
                return carry
            lax.fori_loop(0, NPAIR, pair, 0)
            plsc.subcore_barrier()

            def cphop(i, carry):
                pltpu.sync_copy(
                    acc_sh.at[pl.ds(r0 + i * RB_CP, RB_CP)], buf_v)
                pltpu.sync_copy(
                    buf_v, o_q.at[pl.ds(r0 + i * RB_CP, RB_CP)])
                return carry
            lax.fori_loop(0, N_CP, cphop, 0)
            lax.fori_loop(0, RB_CP, zrow, 0)

        @pl.when(c == 0)
        def _():
            pipeline(h0_hbm, o0_hbm)
            pipeline(h1_hbm, o1_hbm)

        @pl.when(c == 1)
        def _():
            pipeline(h2_hbm, o2_hbm)
            pipeline(h3_hbm, o3_hbm)

    return k(h4[0], h4[1], h4[2], h4[3], src3, dst3, w2)


def kernel(x, edge_index, edge_weight, W, b):
    src = edge_index[0].astype(jnp.int32)
    dst = edge_index[1].astype(jnp.int32)
    pad = E_PAD - N_EDGES
    zi = jnp.zeros((pad,), jnp.int32)
    src3 = jnp.concatenate([src, zi]).reshape(NS, NCHUNK, K)
    dst3 = jnp.concatenate([dst, zi]).reshape(NS, NCHUNK, K)
    w2 = jnp.concatenate(
        [edge_weight, jnp.zeros((pad,), jnp.float32)]).reshape(NS, E_PER_TILE)
    wt4 = jnp.transpose(W.T.reshape(D_IN, 4, DQ), (1, 0, 2))
    h4 = _linear_quarters(x, wt4, b.reshape(4, 1, DQ))
    outs = _sc_aggregate(h4, src3, dst3, w2)
    return jnp.concatenate([o[:N_NODES] for o in outs], axis=1)


# speedup vs baseline: 2.9867x; 1.0040x over previous
import functools

import jax
import jax.numpy as jnp
from jax import lax
from jax.experimental import pallas as pl
from jax.experimental.pallas import tpu as pltpu
from jax.experimental.pallas import tpu_sc as plsc

N_NODES = 10000
N_EDGES = 160000
D_IN = 256
D_OUT = 256
DQ = 64
NC, NS, L = 2, 16, 16
K = 128
E_PAD = 163840
E_PER_TILE = E_PAD // NS
NCHUNK = E_PER_TILE // K
NPAIR = NCHUNK // 2
N_PAD = 10240
ROWS_PER_TILE = N_PAD // NS
RB_CP = 128
N_CP = ROWS_PER_TILE // RB_CP


def _mm_body(x_ref, wt_ref, b_ref, o_ref):
    o_ref[0] = jnp.dot(x_ref[...], wt_ref[0],
                       preferred_element_type=jnp.float32) + b_ref[0]


def _linear_quarters(x, Wt4, b4):
    RB = 1000
    return pl.pallas_call(
        _mm_body,
        grid=(N_NODES // RB, 4),
        in_specs=[
            pl.BlockSpec((RB, D_IN), lambda i, j: (i, 0)),
            pl.BlockSpec((1, D_IN, DQ), lambda i, j: (j, 0, 0)),
            pl.BlockSpec((1, 1, DQ), lambda i, j: (j, 0, 0)),
        ],
        out_specs=pl.BlockSpec((1, RB, DQ), lambda i, j: (j, i, 0)),
        out_shape=jax.ShapeDtypeStruct((4, N_NODES, DQ), jnp.float32),
    )(x, Wt4, b4)


def _sc_aggregate(h4, src3, dst3, w2):
    mesh = plsc.VectorSubcoreMesh(core_axis_name="c", subcore_axis_name="s")

    @functools.partial(
        pl.kernel,
        out_type=[jax.ShapeDtypeStruct((N_PAD, DQ), jnp.float32)] * 4,
        mesh=mesh,
        scratch_types=[
            pltpu.VMEM((NCHUNK, K), jnp.int32),
            pltpu.VMEM((NCHUNK, K), jnp.int32),
            pltpu.VMEM((E_PER_TILE,), jnp.float32),
            pltpu.VMEM((K, DQ), jnp.float32),
            pltpu.VMEM((K, DQ), jnp.float32),
            pltpu.VMEM((K, DQ), jnp.float32),
            pltpu.VMEM((K, DQ), jnp.float32),
            pltpu.VMEM((RB_CP, DQ), jnp.float32),
            pltpu.VMEM_SHARED((N_PAD, DQ), jnp.float32),
            pltpu.SemaphoreType.DMA,
            pltpu.SemaphoreType.DMA,
            pltpu.SemaphoreType.DMA,
            pltpu.SemaphoreType.DMA,
        ],
        compiler_params=pltpu.CompilerParams(use_tc_tiling_on_sc=False),
    )
    def k(h0_hbm, h1_hbm, h2_hbm, h3_hbm, src_hbm, dst_hbm, w_hbm,
          o0_hbm, o1_hbm, o2_hbm, o3_hbm,
          src_v, dst_v, w_v, gbuf0, gbuf1, sbuf0, sbuf1, buf_v, acc_sh,
          gsem0, gsem1, ssem0, ssem1):
        c = lax.axis_index("c")
        s = lax.axis_index("s")
        r0 = s * ROWS_PER_TILE

        pltpu.sync_copy(src_hbm.at[s], src_v)
        pltpu.sync_copy(dst_hbm.at[s], dst_v)
        pltpu.sync_copy(w_hbm.at[s], w_v)

        def zrow(i, carry):
            for j in range(DQ // L):
                buf_v[i, pl.ds(j * L, L)] = jnp.zeros((L,), jnp.float32)
            return carry
        lax.fori_loop(0, RB_CP, zrow, 0)

        def process(t, gbuf, sbuf):
            def mul(g, cc):
                wv16 = w_v[pl.ds(t * K + g * L, L)]
                for e in range(L):
                    wb = wv16[e]
                    i = g * L + e
                    for j in range(DQ // L):
                        sl = pl.ds(j * L, L)
                        sbuf[i, sl] = gbuf[i, sl] * wb
                return cc
            lax.fori_loop(0, K // L, mul, 0)

        def pipeline(h_q, o_q):
            def zhop(i, carry):
                pltpu.sync_copy(
                    buf_v, acc_sh.at[pl.ds(r0 + i * RB_CP, RB_CP)])
                return carry
            lax.fori_loop(0, N_CP, zhop, 0)
            plsc.subcore_barrier()

            pltpu.async_copy(h_q.at[src_v.at[0]], gbuf0, gsem0)
            pltpu.async_copy(h_q.at[src_v.at[1]], gbuf1, gsem1)

            def stage(t, gbuf, sbuf, gsem, ssem, u):
                pltpu.make_async_copy(
                    h_q.at[src_v.at[t]], gbuf, gsem).wait()

                @pl.when(u > 0)
                def _():
                    pltpu.make_async_copy(
                        sbuf, acc_sh.at[dst_v.at[t]], ssem).wait()
                process(t, gbuf, sbuf)

                @pl.when(t + 2 < NCHUNK)
                def _():
                    pltpu.async_copy(h_q.at[src_v.at[t + 2]], gbuf, gsem)
                pltpu.async_copy(sbuf, acc_sh.at[dst_v.at[t]], ssem, add=True)

            def pair(u, carry):
                t0 = 2 * u
                stage(t0, gbuf0, sbuf0, gsem0, ssem0, u)
                stage(t0 + 1, gbuf1, sbuf1, gsem1, ssem1, u)
                return carry
            lax.fori_loop(0, NPAIR, pair, 0)
            pltpu.make_async_copy(
                sbuf0, acc_sh.at[dst_v.at[NCHUNK - 2]], ssem0).wait()
            pltpu.make_async_copy(
                sbuf1, acc_sh.at[dst_v.at[NCHUNK - 1]], ssem1).wait()
            plsc.subcore_barrier()

            def cphop(i, carry):
                pltpu.sync_copy(
                    acc_sh.at[pl.ds(r0 + i * RB_CP, RB_CP)], buf_v)
                pltpu.sync_copy(
                    buf_v, o_q.at[pl.ds(r0 + i * RB_CP, RB_CP)])
                return carry
            lax.fori_loop(0, N_CP, cphop, 0)
            lax.fori_loop(0, RB_CP, zrow, 0)

        @pl.when(c == 0)
        def _():
            pipeline(h0_hbm, o0_hbm)
            pipeline(h1_hbm, o1_hbm)

        @pl.when(c == 1)
        def _():
            pipeline(h2_hbm, o2_hbm)
            pipeline(h3_hbm, o3_hbm)

    return k(h4[0], h4[1], h4[2], h4[3], src3, dst3, w2)


def kernel(x, edge_index, edge_weight, W, b):
    src = edge_index[0].astype(jnp.int32)
    dst = edge_index[1].astype(jnp.int32)
    pad = E_PAD - N_EDGES
    zi = jnp.zeros((pad,), jnp.int32)
    src3 = jnp.concatenate([src, zi]).reshape(NS, NCHUNK, K)
    dst3 = jnp.concatenate([dst, zi]).reshape(NS, NCHUNK, K)
    w2 = jnp.concatenate(
        [edge_weight, jnp.zeros((pad,), jnp.float32)]).reshape(NS, E_PER_TILE)
    wt4 = jnp.transpose(W.T.reshape(D_IN, 4, DQ), (1, 0, 2))
    h4 = _linear_quarters(x, wt4, b.reshape(4, 1, DQ))
    outs = _sc_aggregate(h4, src3, dst3, w2)
    return jnp.concatenate([o[:N_NODES] for o in outs], axis=1)
